# baseline (speedup 1.0000x reference)
import functools

import jax
import jax.numpy as jnp
from jax import lax
from jax.experimental import pallas as pl
from jax.experimental.pallas import tpu as pltpu

_NUM_CLASSES = 80
_NUM_QUERIES = 300
_TOPK = 100
_H = 256
_W = 256
_IMG_H = 1024.0
_IMG_W = 1024.0
_NEG = -3.0e38


def _topk_kernel(mask_cls_ref, mask_box_ref, svec_ref, qvec_ref, lvec_ref,
                 boxes_ref, scratch_ref):
    x = mask_cls_ref[...]
    pad = jnp.full((_NUM_QUERIES, 128 - _NUM_CLASSES), _NEG, jnp.float32)
    scratch_ref[...] = jnp.concatenate([x, pad], axis=1)

    svec_ref[...] = jnp.zeros((1, 128), jnp.float32)
    qvec_ref[...] = jnp.zeros((1, 128), jnp.int32)
    lvec_ref[...] = jnp.zeros((1, 128), jnp.int32)

    rows = lax.broadcasted_iota(jnp.int32, (_NUM_QUERIES, 128), 0)
    cols = lax.broadcasted_iota(jnp.int32, (_NUM_QUERIES, 128), 1)
    flat = rows * 128 + cols
    lane = lax.broadcasted_iota(jnp.int32, (1, 128), 1)

    def body(k, _):
        cur = scratch_ref[...]
        m = jnp.max(cur)
        fidx = jnp.min(jnp.where(cur == m, flat, jnp.int32(2 ** 30)))
        r = fidx // 128
        c = fidx - r * 128
        scratch_ref[...] = jnp.where(flat == fidx, _NEG, cur)
        hit = lane == k
        svec_ref[...] = jnp.where(hit, m, svec_ref[...])
        qvec_ref[...] = jnp.where(hit, r, qvec_ref[...])
        lvec_ref[...] = jnp.where(hit, c, lvec_ref[...])
        return 0

    lax.fori_loop(0, _TOPK, body, 0)

    svec_ref[...] = jax.nn.sigmoid(svec_ref[...])

    qv = qvec_ref[...]
    oh_t = (lax.broadcasted_iota(jnp.int32, (_NUM_QUERIES, 128), 0)
            == qv).astype(jnp.float32)
    mb = lax.dot_general(oh_t, mask_box_ref[...],
                         (((0,), (0,)), ((), ())),
                         preferred_element_type=jnp.float32)
    cx = mb[:, 0:1]
    cy = mb[:, 1:2]
    w = mb[:, 2:3]
    h = mb[:, 3:4]
    xyxy = jnp.concatenate(
        [cx - 0.5 * w, cy - 0.5 * h, cx + 0.5 * w, cy + 0.5 * h], axis=1)
    col4 = lax.broadcasted_iota(jnp.int32, (128, 4), 1)
    scale = jnp.where(col4 % 2 == 0, _IMG_W, _IMG_H)
    boxes_ref[...] = xyxy * scale


def _mask_kernel(qidx_ref, sig_ref, mask_pred_ref, pred_masks_ref, final_ref):
    i = pl.program_id(0)
    mp = mask_pred_ref[0]
    pm = (mp > 0.0).astype(jnp.float32)
    ms = jax.nn.sigmoid(mp)
    num = jnp.sum(ms * pm)
    den = jnp.sum(pm)
    fs = sig_ref[i] * num / (den + 1e-6)
    pred_masks_ref[0] = pm
    final_ref[...] = jnp.full((1, 1, 128), fs, jnp.float32)


def kernel(mask_cls, mask_pred, mask_box):
    svec, qvec, lvec, boxes128 = pl.pallas_call(
        _topk_kernel,
        out_shape=(
            jax.ShapeDtypeStruct((1, 128), jnp.float32),
            jax.ShapeDtypeStruct((1, 128), jnp.int32),
            jax.ShapeDtypeStruct((1, 128), jnp.int32),
            jax.ShapeDtypeStruct((128, 4), jnp.float32),
        ),
        scratch_shapes=[pltpu.VMEM((_NUM_QUERIES, 128), jnp.float32)],
    )(mask_cls, mask_box)

    qidx = qvec[0, :_TOPK]
    sig = svec[0, :_TOPK]

    grid_spec = pltpu.PrefetchScalarGridSpec(
        num_scalar_prefetch=2,
        grid=(_TOPK,),
        in_specs=[
            pl.BlockSpec((1, _H, _W), lambda i, qidx_ref, sig_ref:
                         (qidx_ref[i], 0, 0)),
        ],
        out_specs=[
            pl.BlockSpec((1, _H, _W), lambda i, qidx_ref, sig_ref: (i, 0, 0)),
            pl.BlockSpec((1, 1, 128), lambda i, qidx_ref, sig_ref: (i, 0, 0)),
        ],
    )
    pred_masks, fsout = pl.pallas_call(
        _mask_kernel,
        grid_spec=grid_spec,
        out_shape=(
            jax.ShapeDtypeStruct((_TOPK, _H, _W), jnp.float32),
            jax.ShapeDtypeStruct((_TOPK, 1, 128), jnp.float32),
        ),
    )(qidx, sig, mask_pred)

    final_scores = fsout[:, 0, 0]
    labels = lvec[0, :_TOPK]
    boxes = boxes128[:_TOPK]
    return final_scores, labels, pred_masks, boxes


# speedup vs baseline: 3.1643x; 3.1643x over previous
import functools

import jax
import jax.numpy as jnp
from jax import lax
from jax.experimental import pallas as pl
from jax.experimental.pallas import tpu as pltpu

_NUM_CLASSES = 80
_NUM_QUERIES = 300
_TOPK = 100
_H = 256
_W = 256
_IMG_H = 1024.0
_IMG_W = 1024.0
_NEG = -3.0e38
_QPAD = 304
_BIGI = 2 ** 30
_NBUF = 32


def _topk_kernel(mask_cls_ref, mask_box_ref, svec_ref, qvec_ref, lvec_ref,
                 boxes_ref, m_ref):
    x = mask_cls_ref[...]
    xp = jnp.concatenate(
        [x, jnp.full((_NUM_QUERIES, 128 - _NUM_CLASSES), _NEG, jnp.float32)],
        axis=1)
    xp = jnp.concatenate(
        [xp, jnp.full((_QPAD - _NUM_QUERIES, 128), _NEG, jnp.float32)],
        axis=0)

    ib = lax.bitcast_convert_type(xp, jnp.int32)
    key = ib ^ ((ib >> 31) & jnp.int32(0x7FFFFFFF))

    rows = lax.broadcasted_iota(jnp.int32, (_QPAD, 128), 0)
    cols = lax.broadcasted_iota(jnp.int32, (_QPAD, 128), 1)
    fi = rows * 128 + cols

    def tbody(i, u):
        up = u | (jnp.int32(1) << (31 - i))
        tsig = up ^ jnp.int32(-(2 ** 31))
        cnt = jnp.sum((key >= tsig).astype(jnp.int32))
        return jnp.where(cnt >= _TOPK, up, u)

    u = lax.fori_loop(0, 32, tbody, jnp.int32(0))
    t_key = u ^ jnp.int32(-(2 ** 31))

    gt = key > t_key
    eqm = key == t_key
    need = _TOPK - jnp.sum(gt.astype(jnp.int32))

    def jbody(i, lh):
        lo, hi = lh
        mid = (lo + hi) // 2
        g = jnp.sum((eqm & (fi <= mid)).astype(jnp.int32))
        ok = g >= need
        return jnp.where(ok, lo, mid + 1), jnp.where(ok, mid, hi)

    lo, _ = lax.fori_loop(0, 16, jbody,
                          (jnp.int32(0), jnp.int32(_QPAD * 128 - 1)))
    cand = gt | (eqm & (fi <= lo) & (need > 0))
    m_ref[...] = cand.astype(jnp.int32)

    fif = fi.astype(jnp.float32)
    lane2d = lax.broadcasted_iota(jnp.int32, (128, 128), 1)
    row2d = lax.broadcasted_iota(jnp.int32, (128, 128), 0)
    lower_tri = (row2d <= lane2d).astype(jnp.float32)

    def ccond(st):
        return st[3] > 0

    def cbody(st):
        vals_acc, idx_acc, base, rem = st
        act = m_ref[...] > 0
        fr = jnp.min(jnp.where(act, rows, jnp.int32(_BIGI)), axis=0, keepdims=True)
        sel = act & (rows == fr)
        colval = jnp.sum(jnp.where(sel, xp, 0.0), axis=0, keepdims=True)
        colidx = jnp.sum(jnp.where(sel, fif, 0.0), axis=0, keepdims=True)
        has = (fr < _BIGI).astype(jnp.int32)
        csum = lax.dot_general(has.astype(jnp.float32), lower_tri,
                               (((1,), (0,)), ((), ())),
                               preferred_element_type=jnp.float32
                               ).astype(jnp.int32)
        p = base + csum - has
        p_col = jnp.transpose(p)
        h_col = jnp.transpose(has)
        q1 = ((lane2d == p_col) & (h_col > 0)).astype(jnp.float32)
        vals_acc = vals_acc + lax.dot_general(
            colval, q1, (((1,), (0,)), ((), ())),
            preferred_element_type=jnp.float32,
            precision=lax.Precision.HIGHEST)
        idx_acc = idx_acc + lax.dot_general(
            colidx, q1, (((1,), (0,)), ((), ())),
            preferred_element_type=jnp.float32,
            precision=lax.Precision.HIGHEST)
        m_ref[...] = jnp.where(sel, 0, m_ref[...])
        nh = jnp.sum(has)
        return vals_acc, idx_acc, base + nh, rem - nh

    vals_acc, idx_acc, _, _ = lax.while_loop(
        ccond, cbody,
        (jnp.zeros((1, 128), jnp.float32), jnp.zeros((1, 128), jnp.float32),
         jnp.int32(0), jnp.sum(cand.astype(jnp.int32))))

    lane1 = lax.broadcasted_iota(jnp.int32, (1, 128), 1)
    inlane = lane1 < _TOPK
    va = jnp.where(inlane, vals_acc, _NEG)
    ia = jnp.where(inlane, idx_acc, jnp.float32(2 ** 30))

    va_c = jnp.transpose(va)
    ia_c = jnp.transpose(ia)
    beats = (va_c > va) | ((va_c == va) & (ia_c < ia))
    rank = jnp.sum(beats.astype(jnp.float32), axis=0, keepdims=True)
    r_col = jnp.transpose(rank)
    lane2f = lane2d.astype(jnp.float32)
    q2 = (lane2f == r_col).astype(jnp.float32)
    svals = lax.dot_general(va, q2, (((1,), (0,)), ((), ())),
                            preferred_element_type=jnp.float32,
                            precision=lax.Precision.HIGHEST)
    sidx = lax.dot_general(ia, q2, (((1,), (0,)), ((), ())),
                           preferred_element_type=jnp.float32,
                           precision=lax.Precision.HIGHEST)

    si = sidx.astype(jnp.int32)
    qv = si >> 7
    cv = si & 127
    svec_ref[...] = jax.nn.sigmoid(svals)
    qvec_ref[...] = qv
    lvec_ref[...] = cv.reshape(128)[:_TOPK]

    oh_t = (lax.broadcasted_iota(jnp.int32, (_NUM_QUERIES, 128), 0)
            == qv).astype(jnp.float32)
    mb = lax.dot_general(oh_t, mask_box_ref[...],
                         (((0,), (0,)), ((), ())),
                         preferred_element_type=jnp.float32,
                         precision=lax.Precision.HIGHEST)
    cx = mb[:, 0:1]
    cy = mb[:, 1:2]
    w = mb[:, 2:3]
    h = mb[:, 3:4]
    xyxy = jnp.concatenate(
        [cx - 0.5 * w, cy - 0.5 * h, cx + 0.5 * w, cy + 0.5 * h], axis=1)
    col4 = lax.broadcasted_iota(jnp.int32, (128, 4), 1)
    scale = jnp.where(col4 % 2 == 0, _IMG_W, _IMG_H)
    boxes_ref[...] = (xyxy * scale)[:_TOPK]


def _mask_kernel(qidx_ref, sig_ref, mask_pred_ref, pred_masks_ref,
                 fsout_ref, inbuf, outbuf, num_s, den_s, in_sems, out_sems):
    def in_copy(k, slot):
        return pltpu.make_async_copy(
            mask_pred_ref.at[pl.ds(qidx_ref[0, k], 1)],
            inbuf.at[pl.ds(slot, 1)], in_sems.at[slot])

    def out_copy(k, slot):
        return pltpu.make_async_copy(
            outbuf.at[pl.ds(slot, 1)],
            pred_masks_ref.at[pl.ds(k, 1)], out_sems.at[slot])

    for k in range(_NBUF):
        in_copy(k, k).start()

    def body(k, _):
        slot = k % _NBUF
        in_copy(k, slot).wait()
        mp = inbuf[slot]
        pm = (mp > 0.0).astype(jnp.float32)
        t = jnp.where(mp > 0.0, jax.nn.sigmoid(mp), 0.0)
        ct = jnp.sum(t, axis=0, keepdims=True)
        cp = jnp.sum(pm, axis=0, keepdims=True)
        num_s[pl.ds(k, 1), :] = ct[:, :128] + ct[:, 128:]
        den_s[pl.ds(k, 1), :] = cp[:, :128] + cp[:, 128:]

        @pl.when(k >= _NBUF)
        def _():
            out_copy(k - _NBUF, slot).wait()

        outbuf[slot] = pm
        out_copy(k, slot).start()

        @pl.when(k + _NBUF < _TOPK)
        def _():
            in_copy(k + _NBUF, slot).start()

        return 0

    lax.fori_loop(0, _TOPK, body, 0)

    for j in range(_NBUF):
        out_copy(_TOPK - _NBUF + j, (_TOPK - _NBUF + j) % _NBUF).wait()

    nsum = jnp.sum(num_s[...], axis=1, keepdims=True)
    dsum = jnp.sum(den_s[...], axis=1, keepdims=True)
    sg_col = jnp.transpose(sig_ref[...])[:_TOPK]
    fsout_ref[...] = sg_col * nsum / (dsum + 1e-6)


def kernel(mask_cls, mask_pred, mask_box):
    svec, qvec, labels, boxes = pl.pallas_call(
        _topk_kernel,
        out_shape=(
            jax.ShapeDtypeStruct((1, 128), jnp.float32),
            jax.ShapeDtypeStruct((1, 128), jnp.int32),
            jax.ShapeDtypeStruct((_TOPK,), jnp.int32),
            jax.ShapeDtypeStruct((_TOPK, 4), jnp.float32),
        ),
        scratch_shapes=[
            pltpu.VMEM((_QPAD, 128), jnp.int32),
        ],
    )(mask_cls, mask_box)

    pred_masks, fsout = pl.pallas_call(
        _mask_kernel,
        in_specs=[
            pl.BlockSpec(memory_space=pltpu.SMEM),
            pl.BlockSpec(memory_space=pltpu.VMEM),
            pl.BlockSpec(memory_space=pl.ANY),
        ],
        out_specs=[
            pl.BlockSpec(memory_space=pl.ANY),
            pl.BlockSpec(memory_space=pltpu.VMEM),
        ],
        out_shape=(
            jax.ShapeDtypeStruct((_TOPK, _H, _W), jnp.float32),
            jax.ShapeDtypeStruct((_TOPK, 1), jnp.float32),
        ),
        scratch_shapes=[
            pltpu.VMEM((_NBUF, _H, _W), jnp.float32),
            pltpu.VMEM((_NBUF, _H, _W), jnp.float32),
            pltpu.VMEM((_TOPK, 128), jnp.float32),
            pltpu.VMEM((_TOPK, 128), jnp.float32),
            pltpu.SemaphoreType.DMA((_NBUF,)),
            pltpu.SemaphoreType.DMA((_NBUF,)),
        ],
    )(qvec, svec, mask_pred)

    final_scores = fsout.reshape(_TOPK)
    return final_scores, labels, pred_masks, boxes
